# baseline (speedup 1.0000x reference)
import functools

import jax
import jax.numpy as jnp
from jax import lax
from jax.experimental import pallas as pl
from jax.experimental.pallas import tpu as pltpu
from jax.experimental.pallas import tpu_sc as plsc

C = 26
F = 13
V = 100000
D = 32

NC = 2
NS = 16
NW = NC * NS
L16 = 16
NQ = 4


def _sc_body(T, cat_ref, cont_ref, tab_ref, w_ref, b_ref, out_ref,
             row_v, q_v, qf_v, o_v, wrow_v, brow_v):
    TQ = T // NQ
    ng = TQ // L16
    wid = lax.axis_index("s") * NC + lax.axis_index("c")
    d = wid

    def cat_unit(c, carry):
        pltpu.sync_copy(tab_ref.at[c, d, :], row_v)
        for q in range(NQ):
            pltpu.sync_copy(cat_ref.at[c, pl.ds(q * TQ, TQ)], q_v)
            base = q * TQ

            def grp(g, cc):
                iv = q_v[pl.ds(g * L16, L16)]
                o_v[pl.ds(base + g * L16, L16)] = plsc.load_gather(row_v, [iv])
                return cc

            lax.fori_loop(0, ng, grp, 0, unroll=8)

        pltpu.sync_copy(o_v, out_ref.at[c, d, :])
        return carry

    lax.fori_loop(0, C, cat_unit, 0)

    dsplat = jnp.full((L16,), d, jnp.int32)

    def cont_unit(f, carry):
        pltpu.sync_copy(w_ref.at[f], wrow_v)
        pltpu.sync_copy(b_ref.at[f], brow_v)
        wv = plsc.load_gather(wrow_v, [dsplat])
        bv = plsc.load_gather(brow_v, [dsplat])
        for q in range(NQ):
            pltpu.sync_copy(cont_ref.at[f, pl.ds(q * TQ, TQ)], qf_v)
            base = q * TQ

            def grp(g, cc):
                vv = qf_v[pl.ds(g * L16, L16)]
                o_v[pl.ds(base + g * L16, L16)] = vv * wv + bv
                return cc

            lax.fori_loop(0, ng, grp, 0, unroll=8)

        pltpu.sync_copy(o_v, out_ref.at[C + f, d, :])
        return carry

    lax.fori_loop(0, F, cont_unit, 0)


def kernel(cat, cont, tables, W, b):
    Bd, Ld, Cd = cat.shape
    T = Bd * Ld

    tab_t = tables.transpose(0, 2, 1)
    cat_t = cat.reshape(T, C).T
    cont_t = cont.reshape(T, F).T

    body = functools.partial(_sc_body, T)
    sc_call = pl.kernel(
        body,
        out_type=jax.ShapeDtypeStruct((C + F, D, T), jnp.float32),
        mesh=plsc.VectorSubcoreMesh(core_axis_name="c", subcore_axis_name="s"),
        scratch_types=[
            pltpu.VMEM((V,), jnp.float32),
            pltpu.VMEM((T // NQ,), jnp.int32),
            pltpu.VMEM((T // NQ,), jnp.float32),
            pltpu.VMEM((T,), jnp.float32),
            pltpu.VMEM((D,), jnp.float32),
            pltpu.VMEM((D,), jnp.float32),
        ],
        compiler_params=pltpu.CompilerParams(needs_layout_passes=False),
        name="emb_layer_sc",
    )
    out_t = sc_call(cat_t, cont_t, tab_t, W, b)
    return out_t.transpose(2, 0, 1)
